# speedup vs baseline: 2.4335x; 2.4335x over previous
import functools

import jax
import jax.numpy as jnp
from jax import lax
from jax.experimental import pallas as pl
from jax.experimental.pallas import tpu as pltpu
from jax.experimental.pallas import tpu_sc as plsc

NNODE = 10000
NEDGE = 320000
NPAD = 10240
FEAT = 128
NCLS = 40
NTILES = 32
NCHUNK = 80
CHUNK = 128
EPAD = NTILES * NCHUNK * CHUNK
ROWS_PER_TILE = NPAD // 16


def _spmm_body(y_hbm, src_hbm, dst_hbm, zeros_hbm, out_hbm,
               src_v, dst_v, buf, acc, sem):
    c = lax.axis_index("c")
    s = lax.axis_index("s")
    w = s * 2 + c
    pltpu.sync_copy(src_hbm.at[w], src_v)
    pltpu.sync_copy(dst_hbm.at[w], dst_v)
    pltpu.sync_copy(zeros_hbm, acc.at[pl.ds(s * ROWS_PER_TILE, ROWS_PER_TILE)])
    plsc.subcore_barrier()

    def body(j, carry):
        pltpu.async_copy(y_hbm.at[src_v.at[j]], buf, sem).wait()
        pltpu.sync_copy(buf, acc.at[dst_v.at[j]], add=True)
        return carry

    lax.fori_loop(0, NCHUNK, body, 0)
    plsc.subcore_barrier()
    pltpu.sync_copy(acc.at[pl.ds(s * ROWS_PER_TILE, ROWS_PER_TILE)],
                    out_hbm.at[c, pl.ds(s * ROWS_PER_TILE, ROWS_PER_TILE)])


def _spmm(y, src3, dst3, zeros):
    mesh = plsc.VectorSubcoreMesh(core_axis_name="c", subcore_axis_name="s")
    f = pl.kernel(
        _spmm_body,
        mesh=mesh,
        out_type=jax.ShapeDtypeStruct((2, NPAD, FEAT), jnp.float32),
        scratch_types=[
            pltpu.VMEM((NCHUNK, CHUNK), jnp.int32),
            pltpu.VMEM((NCHUNK, CHUNK), jnp.int32),
            pltpu.VMEM((CHUNK, FEAT), jnp.float32),
            pltpu.VMEM_SHARED((NPAD, FEAT), jnp.float32),
            pltpu.SemaphoreType.DMA,
        ],
    )
    return f(y, src3, dst3, zeros)


def _combine_body(p_ref, o_ref):
    o_ref[...] = p_ref[0] + p_ref[1]


def _combine(p):
    return pl.pallas_call(
        _combine_body,
        out_shape=jax.ShapeDtypeStruct((NPAD, FEAT), jnp.float32),
    )(p)


def _layer_body(k1_ref, k2_ref, k3_ref, k4_ref, w_ref, b_ref, o_ref):
    acc = jnp.dot(k1_ref[...], w_ref[0:128, :], preferred_element_type=jnp.float32)
    acc += jnp.dot(k2_ref[...], w_ref[128:256, :], preferred_element_type=jnp.float32)
    acc += jnp.dot(k3_ref[...], w_ref[256:384, :], preferred_element_type=jnp.float32)
    acc += jnp.dot(k4_ref[...], w_ref[384:512, :], preferred_element_type=jnp.float32)
    o_ref[...] = jnp.maximum(acc + b_ref[...], 0.0)


def _layer(k1, k2, k3, k4, w, b):
    return pl.pallas_call(
        _layer_body,
        out_shape=jax.ShapeDtypeStruct((NPAD, FEAT), jnp.float32),
    )(k1, k2, k3, k4, w, b.reshape(1, FEAT))


def _head_body(h_ref, wc_ref, bc_ref, wo_ref, bo_ref, o_ref):
    c = jnp.tanh(jnp.dot(h_ref[...], wc_ref[...],
                         preferred_element_type=jnp.float32) + bc_ref[...])
    o = jnp.dot(c, wo_ref[...], preferred_element_type=jnp.float32) + bo_ref[...]
    m = jnp.max(o, axis=1, keepdims=True)
    lse = jnp.log(jnp.sum(jnp.exp(o - m), axis=1, keepdims=True)) + m
    o_ref[...] = o - lse


def _head(h, wc, bc, wo, bo):
    return pl.pallas_call(
        _head_body,
        out_shape=jax.ShapeDtypeStruct((NPAD, NCLS), jnp.float32),
    )(h, wc, bc.reshape(1, FEAT), wo, bo.reshape(1, NCLS))


def kernel(x, edge_index, W0, b0, W1, b1, Wc, bc, Wo, bo):
    src = edge_index[0]
    dst = edge_index[1]
    npadextra = EPAD - NEDGE
    src3 = jnp.concatenate(
        [src, jnp.zeros((npadextra,), jnp.int32)]).reshape(NTILES, NCHUNK, CHUNK)
    dst3 = jnp.concatenate(
        [dst, jnp.full((npadextra,), NNODE, jnp.int32)]).reshape(NTILES, NCHUNK, CHUNK)
    zeros = jnp.zeros((ROWS_PER_TILE, FEAT), jnp.float32)
    xp = jnp.concatenate([x, jnp.zeros((NPAD - NNODE, FEAT), jnp.float32)])

    k1 = _combine(_spmm(xp, src3, dst3, zeros))
    k2 = _combine(_spmm(k1, src3, dst3, zeros))
    k3 = _combine(_spmm(k2, src3, dst3, zeros))
    k4 = _combine(_spmm(k3, src3, dst3, zeros))
    h = _layer(k1, k2, k3, k4, W0, b0)

    m1 = _combine(_spmm(h, src3, dst3, zeros))
    m2 = _combine(_spmm(m1, src3, dst3, zeros))
    m3 = _combine(_spmm(m2, src3, dst3, zeros))
    m4 = _combine(_spmm(m3, src3, dst3, zeros))
    h1 = _layer(m1, m2, m3, m4, W1, b1)

    out = _head(h1, Wc, bc, Wo, bo)
    return out[:NNODE]


# speedup vs baseline: 8.3396x; 3.4270x over previous
import functools

import jax
import jax.numpy as jnp
from jax import lax
from jax.experimental import pallas as pl
from jax.experimental.pallas import tpu as pltpu
from jax.experimental.pallas import tpu_sc as plsc

NNODE = 10000
NEDGE = 320000
NPAD = 10240
FEAT = 128
GF = 16
NPASS = 4
NCLS = 40
NBLK = 4
NTPC = 16
JC = 512
NSUP = 40
G = 4
EPT = NSUP * JC
ROWS_T = NPAD // NTPC


def _chain_body(x_hbm, src_hbm, dst_hbm, zeros_hbm, k_hbm,
                src_v, dst_v, b0, b1, b2, b3, pA, pB, semg, sems):
    c = lax.axis_index("c")
    s = lax.axis_index("s")
    rows = pl.ds(s * ROWS_T, ROWS_T)
    bufs = (b0, b1, b2, b3)
    pltpu.sync_copy(src_hbm.at[s], src_v)
    pltpu.sync_copy(dst_hbm.at[s], dst_v)

    for p in range(NPASS):
        col = pl.ds((2 * p + c) * GF, GF)
        pltpu.sync_copy(x_hbm.at[rows, col], pA.at[rows])
        y, acc = pA, pB
        for step in range(NBLK):
            pltpu.sync_copy(zeros_hbm, acc.at[rows])
            plsc.subcore_barrier()

            def body(i, carry, y=y, acc=acc):
                base = i * G
                gs = [pltpu.async_copy(y.at[src_v.at[base + b]], bufs[b], semg)
                      for b in range(G)]
                for t in gs:
                    t.wait()
                ss = [pltpu.async_copy(bufs[b], acc.at[dst_v.at[base + b]],
                                       sems, add=True)
                      for b in range(G)]
                for t in ss:
                    t.wait()
                return carry

            lax.fori_loop(0, NSUP // G, body, 0)
            plsc.subcore_barrier()
            pltpu.sync_copy(acc.at[rows], k_hbm.at[step, rows, col])
            y, acc = acc, y


def _chain(x2d, src4, dst4, zeros):
    mesh = plsc.VectorSubcoreMesh(core_axis_name="c", subcore_axis_name="s")
    f = pl.kernel(
        _chain_body,
        mesh=mesh,
        compiler_params=pltpu.CompilerParams(use_tc_tiling_on_sc=False),
        out_type=jax.ShapeDtypeStruct((NBLK, NPAD, FEAT), jnp.float32),
        scratch_types=[
            pltpu.VMEM((NSUP, JC), jnp.int32),
            pltpu.VMEM((NSUP, JC), jnp.int32),
            pltpu.VMEM((JC, GF), jnp.float32),
            pltpu.VMEM((JC, GF), jnp.float32),
            pltpu.VMEM((JC, GF), jnp.float32),
            pltpu.VMEM((JC, GF), jnp.float32),
            pltpu.VMEM_SHARED((NPAD, GF), jnp.float32),
            pltpu.VMEM_SHARED((NPAD, GF), jnp.float32),
            pltpu.SemaphoreType.DMA,
            pltpu.SemaphoreType.DMA,
        ],
    )
    return f(x2d, src4, dst4, zeros)


def _layer_body(k_ref, w_ref, b_ref, o_ref):
    acc = jnp.broadcast_to(b_ref[...], (NPAD, FEAT))
    for i in range(NBLK):
        acc = acc + jnp.dot(k_ref[i], w_ref[pl.ds(i * FEAT, FEAT), :],
                            preferred_element_type=jnp.float32)
    o_ref[...] = jnp.maximum(acc, 0.0)


def _layer(k_all, w, b):
    return pl.pallas_call(
        _layer_body,
        out_shape=jax.ShapeDtypeStruct((NPAD, FEAT), jnp.float32),
    )(k_all, w, b.reshape(1, FEAT))


def _head_body(h_ref, wc_ref, bc_ref, wo_ref, bo_ref, o_ref):
    z = jnp.dot(h_ref[...], wc_ref[...], preferred_element_type=jnp.float32)
    c = jnp.tanh(z + bc_ref[...])
    o = jnp.dot(c, wo_ref[...], preferred_element_type=jnp.float32) + bo_ref[...]
    m = jnp.max(o, axis=1, keepdims=True)
    lse = jnp.log(jnp.sum(jnp.exp(o - m), axis=1, keepdims=True)) + m
    o_ref[...] = o - lse


def _head(h, wc, bc, wo, bo):
    return pl.pallas_call(
        _head_body,
        out_shape=jax.ShapeDtypeStruct((NPAD, NCLS), jnp.float32),
    )(h, wc, bc.reshape(1, FEAT), wo, bo.reshape(1, NCLS))


def kernel(x, edge_index, W0, b0, W1, b1, Wc, bc, Wo, bo):
    src = edge_index[0]
    dst = edge_index[1]
    per_tile = NEDGE // NTPC
    padn = EPT - per_tile
    pad_src = jnp.broadcast_to(jnp.arange(padn, dtype=jnp.int32) % NNODE,
                               (NTPC, padn))
    pad_dst = jnp.broadcast_to(
        NNODE + (jnp.arange(padn, dtype=jnp.int32) % (NPAD - NNODE)),
        (NTPC, padn))
    src4 = jnp.concatenate([src.reshape(NTPC, per_tile), pad_src],
                           axis=1).reshape(NTPC, NSUP, JC)
    dst4 = jnp.concatenate([dst.reshape(NTPC, per_tile), pad_dst],
                           axis=1).reshape(NTPC, NSUP, JC)
    zeros = jnp.zeros((ROWS_T, GF), jnp.float32)
    xp = jnp.concatenate([x, jnp.zeros((NPAD - NNODE, FEAT), jnp.float32)])

    k_all = _chain(xp, src4, dst4, zeros)
    h = _layer(k_all, W0, b0)
    m_all = _chain(h, src4, dst4, zeros)
    h1 = _layer(m_all, W1, b1)
    out = _head(h1, Wc, bc, Wo, bo)
    return out[:NNODE]


# speedup vs baseline: 9.5819x; 1.1490x over previous
import functools

import jax
import jax.numpy as jnp
from jax import lax
from jax.experimental import pallas as pl
from jax.experimental.pallas import tpu as pltpu
from jax.experimental.pallas import tpu_sc as plsc

NNODE = 10000
NEDGE = 320000
NPAD = 10240
FEAT = 128
GF = 16
NPASS = 4
NCLS = 40
NBLK = 4
NTPC = 16
JC = 512
NSUP = 40
G = 4
EPT = NSUP * JC
ROWS_T = NPAD // NTPC


def _chain_body(x_hbm, src_hbm, dst_hbm, zeros_hbm, k_hbm,
                src_v, dst_v,
                a0, a1, a2, a3, c0, c1, c2, c3, pA, pB, semg, sems):
    c = lax.axis_index("c")
    s = lax.axis_index("s")
    rows = pl.ds(s * ROWS_T, ROWS_T)
    banks = ((a0, a1, a2, a3), (c0, c1, c2, c3))
    nset = NSUP // G
    pltpu.sync_copy(src_hbm.at[s], src_v)
    pltpu.sync_copy(dst_hbm.at[s], dst_v)

    for p in range(NPASS):
        col = pl.ds((2 * p + c) * GF, GF)
        pltpu.sync_copy(x_hbm.at[rows, col], pA.at[rows])
        y, acc = pA, pB
        for step in range(NBLK):
            pltpu.sync_copy(zeros_hbm, acc.at[rows])
            plsc.subcore_barrier()

            def gath(se, bank, y=y):
                for b in range(G):
                    pltpu.async_copy(y.at[src_v.at[se * G + b]],
                                     banks[bank][b], semg)

            def gath_wait(se, bank, y=y):
                for b in range(G):
                    pltpu.make_async_copy(y.at[src_v.at[se * G + b]],
                                          banks[bank][b], semg).wait()

            def scat(se, bank, acc=acc):
                for b in range(G):
                    pltpu.async_copy(banks[bank][b],
                                     acc.at[dst_v.at[se * G + b]], sems,
                                     add=True)

            def scat_wait(se, bank, acc=acc):
                for b in range(G):
                    pltpu.make_async_copy(banks[bank][b],
                                          acc.at[dst_v.at[se * G + b]],
                                          sems).wait()

            gath(0, 0)

            def body(i, carry, gath=gath, gath_wait=gath_wait, scat=scat,
                     scat_wait=scat_wait):
                for b in (0, 1):
                    se = 2 * i + b

                    @pl.when(se >= 1)
                    def _():
                        scat_wait(se - 1, 1 - b)

                    @pl.when(se <= nset - 2)
                    def _():
                        gath(se + 1, 1 - b)

                    gath_wait(se, b)
                    scat(se, b)
                return carry

            lax.fori_loop(0, nset // 2, body, 0)
            scat_wait(nset - 1, 1)
            plsc.subcore_barrier()
            pltpu.sync_copy(acc.at[rows], k_hbm.at[step, rows, col])
            y, acc = acc, y


def _chain(x2d, src4, dst4, zeros):
    mesh = plsc.VectorSubcoreMesh(core_axis_name="c", subcore_axis_name="s")
    f = pl.kernel(
        _chain_body,
        mesh=mesh,
        compiler_params=pltpu.CompilerParams(use_tc_tiling_on_sc=False),
        out_type=jax.ShapeDtypeStruct((NBLK, NPAD, FEAT), jnp.float32),
        scratch_types=[
            pltpu.VMEM((NSUP, JC), jnp.int32),
            pltpu.VMEM((NSUP, JC), jnp.int32),
            pltpu.VMEM((JC, GF), jnp.float32),
            pltpu.VMEM((JC, GF), jnp.float32),
            pltpu.VMEM((JC, GF), jnp.float32),
            pltpu.VMEM((JC, GF), jnp.float32),
            pltpu.VMEM((JC, GF), jnp.float32),
            pltpu.VMEM((JC, GF), jnp.float32),
            pltpu.VMEM((JC, GF), jnp.float32),
            pltpu.VMEM((JC, GF), jnp.float32),
            pltpu.VMEM_SHARED((NPAD, GF), jnp.float32),
            pltpu.VMEM_SHARED((NPAD, GF), jnp.float32),
            pltpu.SemaphoreType.DMA,
            pltpu.SemaphoreType.DMA,
        ],
    )
    return f(x2d, src4, dst4, zeros)


def _layer_body(k_ref, w_ref, b_ref, o_ref):
    acc = jnp.broadcast_to(b_ref[...], (NPAD, FEAT))
    for i in range(NBLK):
        acc = acc + jnp.dot(k_ref[i], w_ref[pl.ds(i * FEAT, FEAT), :],
                            preferred_element_type=jnp.float32)
    o_ref[...] = jnp.maximum(acc, 0.0)


def _layer(k_all, w, b):
    return pl.pallas_call(
        _layer_body,
        out_shape=jax.ShapeDtypeStruct((NPAD, FEAT), jnp.float32),
    )(k_all, w, b.reshape(1, FEAT))


def _head_body(h_ref, wc_ref, bc_ref, wo_ref, bo_ref, o_ref):
    z = jnp.dot(h_ref[...], wc_ref[...], preferred_element_type=jnp.float32)
    c = jnp.tanh(z + bc_ref[...])
    o = jnp.dot(c, wo_ref[...], preferred_element_type=jnp.float32) + bo_ref[...]
    m = jnp.max(o, axis=1, keepdims=True)
    lse = jnp.log(jnp.sum(jnp.exp(o - m), axis=1, keepdims=True)) + m
    o_ref[...] = o - lse


def _head(h, wc, bc, wo, bo):
    return pl.pallas_call(
        _head_body,
        out_shape=jax.ShapeDtypeStruct((NPAD, NCLS), jnp.float32),
    )(h, wc, bc.reshape(1, FEAT), wo, bo.reshape(1, NCLS))


def kernel(x, edge_index, W0, b0, W1, b1, Wc, bc, Wo, bo):
    src = edge_index[0]
    dst = edge_index[1]
    per_tile = NEDGE // NTPC
    padn = EPT - per_tile
    pad_src = jnp.broadcast_to(jnp.arange(padn, dtype=jnp.int32) % NNODE,
                               (NTPC, padn))
    pad_dst = jnp.broadcast_to(
        NNODE + (jnp.arange(padn, dtype=jnp.int32) % (NPAD - NNODE)),
        (NTPC, padn))
    src4 = jnp.concatenate([src.reshape(NTPC, per_tile), pad_src],
                           axis=1).reshape(NTPC, NSUP, JC)
    dst4 = jnp.concatenate([dst.reshape(NTPC, per_tile), pad_dst],
                           axis=1).reshape(NTPC, NSUP, JC)
    zeros = jnp.zeros((ROWS_T, GF), jnp.float32)
    xp = jnp.concatenate([x, jnp.zeros((NPAD - NNODE, FEAT), jnp.float32)])

    k_all = _chain(xp, src4, dst4, zeros)
    h = _layer(k_all, W0, b0)
    m_all = _chain(h, src4, dst4, zeros)
    h1 = _layer(m_all, W1, b1)
    out = _head(h1, Wc, bc, Wo, bo)
    return out[:NNODE]
